# baseline (device time: 429339 ns/iter reference)
import jax
import jax.numpy as jnp
from jax import lax
from jax.experimental import pallas as pl
from jax.experimental.pallas import tpu as pltpu

N_DEV = 32
B = 2
SQ = 128
HQ = 4
DH = 64
SKV = N_DEV * SQ


def kernel(x, Wq, K_ext, V_ext, Wo):
    def body(x_ref, wq_ref, k_ref, v_ref, wo_ref, out_ref,
             kvbuf, stage, copy_sem, send_sems, recv_sems):
        me = lax.axis_index("i")
        left = jnp.mod(me - 1, N_DEV)
        right = jnp.mod(me + 1, N_DEV)

        barrier_sem = pltpu.get_barrier_semaphore()
        for nbr in (left, right):
            pl.semaphore_signal(
                barrier_sem, inc=1,
                device_id=(nbr,), device_id_type=pl.DeviceIdType.MESH,
            )
        pl.semaphore_wait(barrier_sem, 2)

        stage[0] = jnp.transpose(k_ref[...], (0, 2, 1, 3))
        stage[1] = jnp.transpose(v_ref[...], (0, 2, 1, 3))
        own = pltpu.make_async_copy(stage, kvbuf.at[me], copy_sem)
        own.start()
        own.wait()

        for h in range(N_DEV - 1):
            slot = jnp.mod(me - h, N_DEV)
            rdma = pltpu.make_async_remote_copy(
                src_ref=kvbuf.at[slot],
                dst_ref=kvbuf.at[slot],
                send_sem=send_sems.at[h],
                recv_sem=recv_sems.at[h],
                device_id=(right,),
                device_id_type=pl.DeviceIdType.MESH,
            )
            rdma.start()
            rdma.wait()

        q_base = me * SQ
        qi = q_base + lax.broadcasted_iota(jnp.int32, (SQ, SKV), 0)
        ki = lax.broadcasted_iota(jnp.int32, (SQ, SKV), 1)
        mask = (jnp.abs(qi - ki) <= 128) | (ki < 32) | (qi < 32)

        for b in range(B):
            q_all = jax.lax.dot_general(
                x_ref[b], wq_ref[...],
                (((1,), (0,)), ((), ())),
                preferred_element_type=jnp.float32,
            ).reshape(SQ, HQ, DH)
            ctx_heads = []
            for h in range(HQ):
                k_all = kvbuf[:, 0, b, h, :, :].reshape(SKV, DH)
                v_all = kvbuf[:, 1, b, h, :, :].reshape(SKV, DH)
                scores = jax.lax.dot_general(
                    q_all[:, h, :], k_all,
                    (((1,), (1,)), ((), ())),
                    preferred_element_type=jnp.float32,
                ) * 0.125
                scores = jnp.where(mask, scores, -1e9)
                m = jnp.max(scores, axis=-1, keepdims=True)
                w = jnp.exp(scores - m)
                w = w / jnp.sum(w, axis=-1, keepdims=True)
                ctx_heads.append(jax.lax.dot_general(
                    w, v_all,
                    (((1,), (0,)), ((), ())),
                    preferred_element_type=jnp.float32,
                ))
            ctx = jnp.concatenate(ctx_heads, axis=-1)
            out_ref[b] = jax.lax.dot_general(
                ctx, wo_ref[...],
                (((1,), (0,)), ((), ())),
                preferred_element_type=jnp.float32,
            )

    return pl.pallas_call(
        body,
        out_shape=jax.ShapeDtypeStruct((B, SQ, 512), jnp.float32),
        in_specs=[pl.BlockSpec(memory_space=pltpu.VMEM)] * 5,
        out_specs=pl.BlockSpec(memory_space=pltpu.VMEM),
        scratch_shapes=[
            pltpu.VMEM((N_DEV, 2, B, HQ, SQ, DH), jnp.float32),
            pltpu.VMEM((2, B, HQ, SQ, DH), jnp.float32),
            pltpu.SemaphoreType.DMA,
            pltpu.SemaphoreType.DMA((N_DEV - 1,)),
            pltpu.SemaphoreType.DMA((N_DEV - 1,)),
        ],
        compiler_params=pltpu.CompilerParams(
            collective_id=0,
            vmem_limit_bytes=100 * 1024 * 1024,
        ),
    )(x, Wq, K_ext, V_ext, Wo)


# device time: 131553 ns/iter; 3.2636x vs baseline; 3.2636x over previous
import jax
import jax.numpy as jnp
from jax import lax
from jax.experimental import pallas as pl
from jax.experimental.pallas import tpu as pltpu

N_DEV = 32
B = 2
SQ = 128
HQ = 4
DH = 64
G = 32
DM = 512
DQ = 256


def kernel(x, Wq, K_ext, V_ext, Wo):
    def body(x_ref, wq_ref, k_ref, v_ref, wo_ref, out_ref,
             stage, nbuf, bc, pstage, pbuf,
             nsend, nrecv, bsend, brecv, psend, precv):
        me = lax.axis_index("i")
        left = jnp.mod(me - 1, N_DEV)
        right = jnp.mod(me + 1, N_DEV)

        barrier_sem = pltpu.get_barrier_semaphore()
        for nbr in (left, right):
            pl.semaphore_signal(
                barrier_sem, inc=1,
                device_id=(nbr,), device_id_type=pl.DeviceIdType.MESH,
            )

        @pl.when(me != 0)
        def _():
            pl.semaphore_signal(
                barrier_sem, inc=1,
                device_id=(0,), device_id_type=pl.DeviceIdType.MESH,
            )
            pl.semaphore_wait(barrier_sem, 2)

        @pl.when(me == 0)
        def _():
            pl.semaphore_wait(barrier_sem, N_DEV - 1 + 2)

        stage[0] = jnp.transpose(k_ref[...], (0, 2, 1, 3))
        stage[1] = jnp.transpose(v_ref[...], (0, 2, 1, 3))

        send_l = pltpu.make_async_remote_copy(
            src_ref=stage, dst_ref=nbuf.at[1],
            send_sem=nsend.at[0], recv_sem=nrecv.at[1],
            device_id=(left,), device_id_type=pl.DeviceIdType.MESH,
        )
        send_r = pltpu.make_async_remote_copy(
            src_ref=stage, dst_ref=nbuf.at[0],
            send_sem=nsend.at[1], recv_sem=nrecv.at[0],
            device_id=(right,), device_id_type=pl.DeviceIdType.MESH,
        )

        @pl.when(me != 0)
        def _():
            send_l.start()

        @pl.when(me != N_DEV - 1)
        def _():
            send_r.start()

        q_bh = []
        for b in range(B):
            q_all = jax.lax.dot_general(
                x_ref[b], wq_ref[...],
                (((1,), (0,)), ((), ())),
                preferred_element_type=jnp.float32,
            )
            q_bh.append([q_all[:, h * DH:(h + 1) * DH] for h in range(HQ)])

        @pl.when(me == 0)
        def _():
            for b in range(B):
                qg = jnp.stack(
                    [q_bh[b][h][0:G, :] for h in range(HQ)], axis=0
                )
                bc[0, b] = qg
            bc[1] = stage[0, :, :, 0:G, :]
            bc[2] = stage[1, :, :, 0:G, :]
            for s in range(1, N_DEV):
                r = pltpu.make_async_remote_copy(
                    src_ref=bc, dst_ref=bc,
                    send_sem=bsend.at[s - 1], recv_sem=brecv,
                    device_id=(s,), device_id_type=pl.DeviceIdType.MESH,
                )
                r.start()

        bc_desc = pltpu.make_async_remote_copy(
            src_ref=bc, dst_ref=bc,
            send_sem=bsend.at[0], recv_sem=brecv,
            device_id=(0,), device_id_type=pl.DeviceIdType.MESH,
        )

        @pl.when(me != 0)
        def _():
            bc_desc.wait_recv()

        for b in range(B):
            for h in range(HQ):
                qg = bc[0, b, h]
                sc = jax.lax.dot_general(
                    qg, stage[0, b, h],
                    (((1,), (1,)), ((), ())),
                    preferred_element_type=jnp.float32,
                ) * 0.125
                m_s = jnp.max(sc, axis=-1)
                e = jnp.exp(sc - m_s[:, None])
                l_s = jnp.sum(e, axis=-1)
                u_s = jax.lax.dot_general(
                    e, stage[1, b, h],
                    (((1,), (0,)), ((), ())),
                    preferred_element_type=jnp.float32,
                )
                row32 = jnp.concatenate([m_s, l_s])[None, :]
                pstage[b, h] = jnp.concatenate([u_s, row32], axis=0)

        p_desc = pltpu.make_async_remote_copy(
            src_ref=pstage, dst_ref=pbuf.at[me],
            send_sem=psend, recv_sem=precv.at[me],
            device_id=(0,), device_id_type=pl.DeviceIdType.MESH,
        )

        @pl.when(me != 0)
        def _():
            p_desc.start()

        @pl.when(me == 0)
        def _():
            pbuf[0] = pstage[...]

        @pl.when(me != 0)
        def _():
            send_l.wait_send()

        @pl.when(me != N_DEV - 1)
        def _():
            send_r.wait_send()

        recv_l = pltpu.make_async_remote_copy(
            src_ref=stage, dst_ref=nbuf.at[0],
            send_sem=nsend.at[0], recv_sem=nrecv.at[0],
            device_id=(left,), device_id_type=pl.DeviceIdType.MESH,
        )
        recv_r = pltpu.make_async_remote_copy(
            src_ref=stage, dst_ref=nbuf.at[1],
            send_sem=nsend.at[1], recv_sem=nrecv.at[1],
            device_id=(right,), device_id_type=pl.DeviceIdType.MESH,
        )

        @pl.when(me != 0)
        def _():
            recv_l.wait_recv()

        @pl.when(me != N_DEV - 1)
        def _():
            recv_r.wait_recv()

        NK = 3 * SQ + G
        qi = me * SQ + lax.broadcasted_iota(jnp.int32, (SQ, NK), 0)
        kcol = lax.broadcasted_iota(jnp.int32, (SQ, NK), 1)
        me_v = jnp.full((SQ, NK), me, dtype=jnp.int32)
        ki = jnp.where(
            kcol < 3 * SQ, (me - 1) * SQ + kcol, kcol - 3 * SQ
        )
        ok_left = (kcol >= SQ) | (me_v > 0)
        in_right = (kcol >= 2 * SQ) & (kcol < 3 * SQ)
        ok_right = (~in_right) | (me_v < N_DEV - 1)
        ok_glob = (kcol < 3 * SQ) | (me_v >= 2)
        window = (jnp.abs(qi - ki) <= 128) | (ki < G)
        mask = ok_left & ok_right & ok_glob & window

        for b in range(B):
            ctx_heads = []
            for h in range(HQ):
                k_loc = jnp.concatenate([
                    nbuf[0, 0, b, h], stage[0, b, h], nbuf[1, 0, b, h],
                    bc[1, b, h],
                ], axis=0)
                v_loc = jnp.concatenate([
                    nbuf[0, 1, b, h], stage[1, b, h], nbuf[1, 1, b, h],
                    bc[2, b, h],
                ], axis=0)
                sc = jax.lax.dot_general(
                    q_bh[b][h], k_loc,
                    (((1,), (1,)), ((), ())),
                    preferred_element_type=jnp.float32,
                ) * 0.125
                sc = jnp.where(mask, sc, -1e9)
                mx = jnp.max(sc, axis=-1, keepdims=True)
                w = jnp.exp(sc - mx)
                w = w / jnp.sum(w, axis=-1, keepdims=True)
                ctx_heads.append(jax.lax.dot_general(
                    w, v_loc,
                    (((1,), (0,)), ((), ())),
                    preferred_element_type=jnp.float32,
                ))
            ctx = jnp.concatenate(ctx_heads, axis=-1)
            out_ref[b] = jax.lax.dot_general(
                ctx, wo_ref[...],
                (((1,), (0,)), ((), ())),
                preferred_element_type=jnp.float32,
            )

        @pl.when(me == 0)
        def _():
            for s in range(1, N_DEV):
                pltpu.make_async_remote_copy(
                    src_ref=pstage, dst_ref=pbuf.at[s],
                    send_sem=psend, recv_sem=precv.at[s],
                    device_id=(0,), device_id_type=pl.DeviceIdType.MESH,
                ).wait_recv()
            for b in range(B):
                glob_heads = []
                for h in range(HQ):
                    U = pbuf[:, b, h, 0:G, :]
                    ML = pbuf[:, b, h, G, :]
                    m_s = ML[:, 0:G]
                    l_s = ML[:, G:2 * G]
                    m = jnp.max(m_s, axis=0)
                    alpha = jnp.exp(m_s - m[None, :])
                    l = jnp.sum(l_s * alpha, axis=0)
                    u = jnp.sum(U * alpha[:, :, None], axis=0)
                    glob_heads.append(u / l[:, None])
                ctx_g = jnp.concatenate(glob_heads, axis=-1)
                out_ref[b, 0:G, :] = jax.lax.dot_general(
                    ctx_g, wo_ref[...],
                    (((1,), (0,)), ((), ())),
                    preferred_element_type=jnp.float32,
                )

        @pl.when(me != 0)
        def _():
            p_desc.wait_send()

        @pl.when(me == 0)
        def _():
            for s in range(1, N_DEV):
                pltpu.make_async_remote_copy(
                    src_ref=bc, dst_ref=bc,
                    send_sem=bsend.at[s - 1], recv_sem=brecv,
                    device_id=(s,), device_id_type=pl.DeviceIdType.MESH,
                ).wait_send()

    return pl.pallas_call(
        body,
        out_shape=jax.ShapeDtypeStruct((B, SQ, DM), jnp.float32),
        in_specs=[pl.BlockSpec(memory_space=pltpu.VMEM)] * 5,
        out_specs=pl.BlockSpec(memory_space=pltpu.VMEM),
        scratch_shapes=[
            pltpu.VMEM((2, B, HQ, SQ, DH), jnp.float32),
            pltpu.VMEM((2, 2, B, HQ, SQ, DH), jnp.float32),
            pltpu.VMEM((3, B, HQ, G, DH), jnp.float32),
            pltpu.VMEM((B, HQ, G + 1, 2 * G), jnp.float32),
            pltpu.VMEM((N_DEV, B, HQ, G + 1, 2 * G), jnp.float32),
            pltpu.SemaphoreType.DMA((2,)),
            pltpu.SemaphoreType.DMA((2,)),
            pltpu.SemaphoreType.DMA((N_DEV - 1,)),
            pltpu.SemaphoreType.DMA,
            pltpu.SemaphoreType.DMA,
            pltpu.SemaphoreType.DMA((N_DEV,)),
        ],
        compiler_params=pltpu.CompilerParams(
            collective_id=0,
            vmem_limit_bytes=100 * 1024 * 1024,
        ),
    )(x, Wq, K_ext, V_ext, Wo)


# device time: 119116 ns/iter; 3.6044x vs baseline; 1.1044x over previous
import jax
import jax.numpy as jnp
from jax import lax
from jax.experimental import pallas as pl
from jax.experimental.pallas import tpu as pltpu

N_DEV = 32
B = 2
SQ = 128
HQ = 4
DH = 64
G = 32
DM = 512
DQ = 256


def kernel(x, Wq, K_ext, V_ext, Wo):
    def body(x_ref, wq_ref, k_ref, v_ref, wo_ref, out_ref,
             stage, nbuf, bcq, bckv, pstage, pbuf,
             nsend, nrecv, bsend_q, bsend_kv, brecv_q, brecv_kv,
             psend, precv):
        me = lax.axis_index("i")
        left = jnp.mod(me - 1, N_DEV)
        right = jnp.mod(me + 1, N_DEV)

        barrier_sem = pltpu.get_barrier_semaphore()
        for nbr in (left, right):
            pl.semaphore_signal(
                barrier_sem, inc=1,
                device_id=(nbr,), device_id_type=pl.DeviceIdType.MESH,
            )

        @pl.when(me != 0)
        def _():
            pl.semaphore_signal(
                barrier_sem, inc=1,
                device_id=(0,), device_id_type=pl.DeviceIdType.MESH,
            )
            pl.semaphore_wait(barrier_sem, 2)

        @pl.when(me == 0)
        def _():
            pl.semaphore_wait(barrier_sem, N_DEV - 1 + 2)

        @pl.when(me == 0)
        def _():
            for b in range(B):
                qg_all = jax.lax.dot_general(
                    x_ref[b, 0:G, :], wq_ref[...],
                    (((1,), (0,)), ((), ())),
                    preferred_element_type=jnp.float32,
                )
                bcq[b] = jnp.stack(
                    [qg_all[:, h * DH:(h + 1) * DH] for h in range(HQ)],
                    axis=0,
                )
            for s in range(1, N_DEV):
                pltpu.make_async_remote_copy(
                    src_ref=bcq, dst_ref=bcq,
                    send_sem=bsend_q.at[s - 1], recv_sem=brecv_q,
                    device_id=(s,), device_id_type=pl.DeviceIdType.MESH,
                ).start()

        stage[0] = jnp.transpose(k_ref[...], (0, 2, 1, 3))
        stage[1] = jnp.transpose(v_ref[...], (0, 2, 1, 3))

        @pl.when(me == 0)
        def _():
            bckv[0] = stage[0, :, :, 0:G, :]
            bckv[1] = stage[1, :, :, 0:G, :]
            for s in range(1, N_DEV):
                pltpu.make_async_remote_copy(
                    src_ref=bckv, dst_ref=bckv,
                    send_sem=bsend_kv.at[s - 1], recv_sem=brecv_kv,
                    device_id=(s,), device_id_type=pl.DeviceIdType.MESH,
                ).start()

        send_l = pltpu.make_async_remote_copy(
            src_ref=stage, dst_ref=nbuf.at[1],
            send_sem=nsend.at[0], recv_sem=nrecv.at[1],
            device_id=(left,), device_id_type=pl.DeviceIdType.MESH,
        )
        send_r = pltpu.make_async_remote_copy(
            src_ref=stage, dst_ref=nbuf.at[0],
            send_sem=nsend.at[1], recv_sem=nrecv.at[0],
            device_id=(right,), device_id_type=pl.DeviceIdType.MESH,
        )

        @pl.when(me != 0)
        def _():
            send_l.start()

        @pl.when(me != N_DEV - 1)
        def _():
            send_r.start()

        bcq_desc = pltpu.make_async_remote_copy(
            src_ref=bcq, dst_ref=bcq,
            send_sem=bsend_q.at[0], recv_sem=brecv_q,
            device_id=(0,), device_id_type=pl.DeviceIdType.MESH,
        )

        @pl.when(me != 0)
        def _():
            bcq_desc.wait_recv()

        for b in range(B):
            for h in range(HQ):
                qg = bcq[b, h]
                sc = jax.lax.dot_general(
                    qg, stage[0, b, h],
                    (((1,), (1,)), ((), ())),
                    preferred_element_type=jnp.float32,
                ) * 0.125
                m_s = jnp.max(sc, axis=-1)
                e = jnp.exp(sc - m_s[:, None])
                l_s = jnp.sum(e, axis=-1)
                u_s = jax.lax.dot_general(
                    e, stage[1, b, h],
                    (((1,), (0,)), ((), ())),
                    preferred_element_type=jnp.float32,
                )
                row32 = jnp.concatenate([m_s, l_s])[None, :]
                pstage[b, h] = jnp.concatenate([u_s, row32], axis=0)

        p_desc = pltpu.make_async_remote_copy(
            src_ref=pstage, dst_ref=pbuf.at[me],
            send_sem=psend, recv_sem=precv.at[me],
            device_id=(0,), device_id_type=pl.DeviceIdType.MESH,
        )

        @pl.when(me != 0)
        def _():
            p_desc.start()

        @pl.when(me == 0)
        def _():
            pbuf[0] = pstage[...]

        q_bh = []
        for b in range(B):
            q_all = jax.lax.dot_general(
                x_ref[b], wq_ref[...],
                (((1,), (0,)), ((), ())),
                preferred_element_type=jnp.float32,
            )
            q_bh.append([q_all[:, h * DH:(h + 1) * DH] for h in range(HQ)])

        @pl.when(me != 0)
        def _():
            send_l.wait_send()

        @pl.when(me != N_DEV - 1)
        def _():
            send_r.wait_send()

        bckv_desc = pltpu.make_async_remote_copy(
            src_ref=bckv, dst_ref=bckv,
            send_sem=bsend_kv.at[0], recv_sem=brecv_kv,
            device_id=(0,), device_id_type=pl.DeviceIdType.MESH,
        )

        @pl.when(me != 0)
        def _():
            bckv_desc.wait_recv()

        recv_l = pltpu.make_async_remote_copy(
            src_ref=stage, dst_ref=nbuf.at[0],
            send_sem=nsend.at[0], recv_sem=nrecv.at[0],
            device_id=(left,), device_id_type=pl.DeviceIdType.MESH,
        )
        recv_r = pltpu.make_async_remote_copy(
            src_ref=stage, dst_ref=nbuf.at[1],
            send_sem=nsend.at[1], recv_sem=nrecv.at[1],
            device_id=(right,), device_id_type=pl.DeviceIdType.MESH,
        )

        @pl.when(me != 0)
        def _():
            recv_l.wait_recv()

        @pl.when(me != N_DEV - 1)
        def _():
            recv_r.wait_recv()

        NK = 3 * SQ + G
        qi = me * SQ + lax.broadcasted_iota(jnp.int32, (SQ, NK), 0)
        kcol = lax.broadcasted_iota(jnp.int32, (SQ, NK), 1)
        me_v = jnp.full((SQ, NK), me, dtype=jnp.int32)
        ki = jnp.where(
            kcol < 3 * SQ, (me - 1) * SQ + kcol, kcol - 3 * SQ
        )
        ok_left = (kcol >= SQ) | (me_v > 0)
        in_right = (kcol >= 2 * SQ) & (kcol < 3 * SQ)
        ok_right = (~in_right) | (me_v < N_DEV - 1)
        ok_glob = (kcol < 3 * SQ) | (me_v >= 2)
        window = (jnp.abs(qi - ki) <= 128) | (ki < G)
        mask = ok_left & ok_right & ok_glob & window

        for b in range(B):
            ctx_heads = []
            for h in range(HQ):
                k_loc = jnp.concatenate([
                    nbuf[0, 0, b, h], stage[0, b, h], nbuf[1, 0, b, h],
                    bckv[0, b, h],
                ], axis=0)
                v_loc = jnp.concatenate([
                    nbuf[0, 1, b, h], stage[1, b, h], nbuf[1, 1, b, h],
                    bckv[1, b, h],
                ], axis=0)
                sc = jax.lax.dot_general(
                    q_bh[b][h], k_loc,
                    (((1,), (1,)), ((), ())),
                    preferred_element_type=jnp.float32,
                ) * 0.125
                sc = jnp.where(mask, sc, -1e9)
                mx = jnp.max(sc, axis=-1, keepdims=True)
                w = jnp.exp(sc - mx)
                w = w / jnp.sum(w, axis=-1, keepdims=True)
                ctx_heads.append(jax.lax.dot_general(
                    w, v_loc,
                    (((1,), (0,)), ((), ())),
                    preferred_element_type=jnp.float32,
                ))
            ctx = jnp.concatenate(ctx_heads, axis=-1)
            out_ref[b] = jax.lax.dot_general(
                ctx, wo_ref[...],
                (((1,), (0,)), ((), ())),
                preferred_element_type=jnp.float32,
            )

        @pl.when(me == 0)
        def _():
            for s in range(1, N_DEV):
                pltpu.make_async_remote_copy(
                    src_ref=pstage, dst_ref=pbuf.at[s],
                    send_sem=psend, recv_sem=precv.at[s],
                    device_id=(0,), device_id_type=pl.DeviceIdType.MESH,
                ).wait_recv()
            for b in range(B):
                glob_heads = []
                for h in range(HQ):
                    U = pbuf[:, b, h, 0:G, :]
                    ML = pbuf[:, b, h, G, :]
                    m_s = ML[:, 0:G]
                    l_s = ML[:, G:2 * G]
                    m = jnp.max(m_s, axis=0)
                    alpha = jnp.exp(m_s - m[None, :])
                    l = jnp.sum(l_s * alpha, axis=0)
                    u = jnp.sum(U * alpha[:, :, None], axis=0)
                    glob_heads.append(u / l[:, None])
                ctx_g = jnp.concatenate(glob_heads, axis=-1)
                out_ref[b, 0:G, :] = jax.lax.dot_general(
                    ctx_g, wo_ref[...],
                    (((1,), (0,)), ((), ())),
                    preferred_element_type=jnp.float32,
                )

        @pl.when(me != 0)
        def _():
            p_desc.wait_send()

        @pl.when(me == 0)
        def _():
            for s in range(1, N_DEV):
                pltpu.make_async_remote_copy(
                    src_ref=bcq, dst_ref=bcq,
                    send_sem=bsend_q.at[s - 1], recv_sem=brecv_q,
                    device_id=(s,), device_id_type=pl.DeviceIdType.MESH,
                ).wait_send()
                pltpu.make_async_remote_copy(
                    src_ref=bckv, dst_ref=bckv,
                    send_sem=bsend_kv.at[s - 1], recv_sem=brecv_kv,
                    device_id=(s,), device_id_type=pl.DeviceIdType.MESH,
                ).wait_send()

    return pl.pallas_call(
        body,
        out_shape=jax.ShapeDtypeStruct((B, SQ, DM), jnp.float32),
        in_specs=[pl.BlockSpec(memory_space=pltpu.VMEM)] * 5,
        out_specs=pl.BlockSpec(memory_space=pltpu.VMEM),
        scratch_shapes=[
            pltpu.VMEM((2, B, HQ, SQ, DH), jnp.float32),
            pltpu.VMEM((2, 2, B, HQ, SQ, DH), jnp.float32),
            pltpu.VMEM((B, HQ, G, DH), jnp.float32),
            pltpu.VMEM((2, B, HQ, G, DH), jnp.float32),
            pltpu.VMEM((B, HQ, G + 1, 2 * G), jnp.float32),
            pltpu.VMEM((N_DEV, B, HQ, G + 1, 2 * G), jnp.float32),
            pltpu.SemaphoreType.DMA((2,)),
            pltpu.SemaphoreType.DMA((2,)),
            pltpu.SemaphoreType.DMA((N_DEV - 1,)),
            pltpu.SemaphoreType.DMA((N_DEV - 1,)),
            pltpu.SemaphoreType.DMA,
            pltpu.SemaphoreType.DMA,
            pltpu.SemaphoreType.DMA,
            pltpu.SemaphoreType.DMA((N_DEV,)),
        ],
        compiler_params=pltpu.CompilerParams(
            collective_id=0,
            vmem_limit_bytes=100 * 1024 * 1024,
        ),
    )(x, Wq, K_ext, V_ext, Wo)
